# baseline (speedup 1.0000x reference)
import functools

import jax
import jax.numpy as jnp
from jax import lax
from jax.experimental import pallas as pl
from jax.experimental.pallas import tpu as pltpu
from jax.experimental.pallas import tpu_sc as plsc

D = 512

_INFO = plsc.get_sparse_core_info()
NC = _INFO.num_cores
NS = _INFO.num_subcores
NW = NC * NS

CHUNK = 32
NBUF = 7


def _gather_body(n_chunks, offset, ids_hbm, table_hbm, out_hbm, idx_v, rows_v, sem_g, sem_s):
    b_per_w = n_chunks * CHUNK
    wid = lax.axis_index("s") * NC + lax.axis_index("c")
    base = wid * b_per_w
    pltpu.sync_copy(ids_hbm.at[pl.ds(offset + base, b_per_w)], idx_v)

    def start_gather(c):
        return pltpu.async_copy(
            table_hbm.at[idx_v.at[pl.ds(c * CHUNK, CHUNK)]],
            rows_v.at[c % NBUF],
            sem_g,
        )

    gat = {c: start_gather(c) for c in range(min(NBUF - 1, n_chunks))}
    sca = {}
    for c in range(n_chunks):
        gat.pop(c).wait()
        if c >= 1:
            sca.pop(c - 1).wait()
        if c + NBUF - 1 < n_chunks:
            gat[c + NBUF - 1] = start_gather(c + NBUF - 1)
        sca[c] = pltpu.async_copy(
            rows_v.at[c % NBUF],
            out_hbm.at[pl.ds(base + c * CHUNK, CHUNK)],
            sem_s,
        )
    sca.pop(n_chunks - 1).wait()


def _sc_gather(ids, table, offset, n_tok):
    b_per_w = n_tok // NW
    n_chunks = b_per_w // CHUNK
    mesh = plsc.VectorSubcoreMesh(core_axis_name="c", subcore_axis_name="s")
    k = pl.kernel(
        functools.partial(_gather_body, n_chunks, offset),
        out_type=jax.ShapeDtypeStruct((n_tok, D), jnp.float32),
        mesh=mesh,
        scratch_types=[
            pltpu.VMEM((n_chunks * CHUNK,), jnp.int32),
            pltpu.VMEM((NBUF, CHUNK, D), jnp.float32),
            pltpu.SemaphoreType.DMA,
            pltpu.SemaphoreType.DMA,
        ],
    )
    return k(ids, table)


def _mlp_body(x_ref, w1_ref, b1_ref, w2_ref, b2_ref, o_ref):
    x = x_ref[...]
    h = jnp.dot(x, w1_ref[...], preferred_element_type=jnp.float32)
    h = jnp.maximum(h + b1_ref[...], 0.0)
    o = jnp.dot(h, w2_ref[...], preferred_element_type=jnp.float32)
    o_ref[...] = o + b2_ref[...]


def _mlp_body_alias(x_ref, w1_ref, b1_ref, w2_ref, b2_ref, prev_ref, o_ref):
    del prev_ref
    _mlp_body(x_ref, w1_ref, b1_ref, w2_ref, b2_ref, o_ref)


_WSPECS = [
    pl.BlockSpec((D, D), lambda i: (0, 0)),
    pl.BlockSpec((1, D), lambda i: (0, 0)),
    pl.BlockSpec((D, D), lambda i: (0, 0)),
    pl.BlockSpec((1, D), lambda i: (0, 0)),
]


def _tc_mlp_part(x, w1, b1, w2, b2, n_tok, off_blk, prev, block_t=2048):
    nblk = x.shape[0] // block_t
    x_spec = pl.BlockSpec((block_t, D), lambda i: (i, 0))
    out_spec = pl.BlockSpec((block_t, D), lambda i: (i + off_blk, 0))
    if prev is None:
        return pl.pallas_call(
            _mlp_body,
            grid=(nblk,),
            in_specs=[x_spec] + _WSPECS,
            out_specs=out_spec,
            out_shape=jax.ShapeDtypeStruct((n_tok, D), jnp.float32),
        )(x, w1, b1.reshape(1, D), w2, b2.reshape(1, D))
    return pl.pallas_call(
        _mlp_body_alias,
        grid=(nblk,),
        in_specs=[x_spec] + _WSPECS + [pl.BlockSpec(memory_space=pl.ANY)],
        out_specs=out_spec,
        out_shape=jax.ShapeDtypeStruct((n_tok, D), jnp.float32),
        input_output_aliases={5: 0},
    )(x, w1, b1.reshape(1, D), w2, b2.reshape(1, D), prev)


CHUNK_SIZES = (16384, 16384)
BLOCK_T = 4096


def kernel(input_ids, emb_table, W1, b1, W2, b2):
    B, S = input_ids.shape
    ids = input_ids.reshape(-1).astype(jnp.int32)
    n_tok = ids.shape[0]
    offs = [sum(CHUNK_SIZES[:k]) for k in range(len(CHUNK_SIZES))]
    xs = [
        _sc_gather(ids, emb_table, offs[k], ct)
        for k, ct in enumerate(CHUNK_SIZES)
    ]
    out = None
    for k, ct in enumerate(CHUNK_SIZES):
        out = _tc_mlp_part(
            xs[k], W1, b1, W2, b2, n_tok,
            off_blk=offs[k] // BLOCK_T, prev=out, block_t=BLOCK_T,
        )
    return out.reshape(B, S, D)
